# Initial kernel scaffold; baseline (speedup 1.0000x reference)
import jax
import jax.numpy as jnp
from jax.experimental import pallas as pl


def kernel(x, codebook):
    raise NotImplementedError("write your pallas kernel here")


# speedup vs baseline: 1.2338x; 1.2338x over previous
import jax
import jax.numpy as jnp
from jax.experimental import pallas as pl
from jax.experimental.pallas import tpu as pltpu

NV = 8192
TAU = 2.0


def _vq_body(x_ref, cb_ref, g_ref, q_ref, p_ref):
    x = x_ref[...]
    cb = cb_ref[...]
    x2 = jnp.sum(x * x, axis=1, keepdims=True)
    c2 = jnp.sum(cb * cb, axis=1)[None, :]
    xc = jax.lax.dot_general(
        x, cb, (((1,), (1,)), ((), ())),
        preferred_element_type=jnp.float32)
    d2 = jnp.maximum(x2 + c2 - 2.0 * xc, 1e-12)
    s = (g_ref[...] - jnp.sqrt(d2)) * (1.0 / TAU)
    m = jnp.max(s, axis=1, keepdims=True)
    e = jnp.exp(s - m)
    p = e * (1.0 / jnp.sum(e, axis=1, keepdims=True))
    p_ref[...] = p
    q_ref[...] = jnp.dot(p, cb, preferred_element_type=jnp.float32)


def kernel(x, codebook):
    b, t, d = x.shape
    n = b * t
    xf = x.reshape(n, d)
    g = jax.random.gumbel(jax.random.key(42), (n, NV), jnp.float32)
    br = 256
    q, p = pl.pallas_call(
        _vq_body,
        grid=(n // br,),
        in_specs=[
            pl.BlockSpec((br, d), lambda i: (i, 0)),
            pl.BlockSpec((NV, d), lambda i: (0, 0)),
            pl.BlockSpec((br, NV), lambda i: (i, 0)),
        ],
        out_specs=[
            pl.BlockSpec((br, d), lambda i: (i, 0)),
            pl.BlockSpec((br, NV), lambda i: (i, 0)),
        ],
        out_shape=[
            jax.ShapeDtypeStruct((n, d), jnp.float32),
            jax.ShapeDtypeStruct((n, NV), jnp.float32),
        ],
    )(xf, codebook, g)
    return q.reshape(b, t, d), p.reshape(b, t, NV)
